# baseline (speedup 1.0000x reference)
import functools

import jax
import jax.numpy as jnp
from jax import lax
from jax.experimental import pallas as pl
from jax.experimental.pallas import tpu as pltpu
from jax.experimental.pallas import tpu_sc as plsc

NC = 2
NS = 16
LANES = 16
NW = NC * NS

BATCH = 4096
SEQ = 200
DIM = 64
ROWS_PER_W = BATCH // NW
IDX_CHUNK = 100


def _body(inputs_hbm, tok_hbm, pos_hbm, out_hbm, idx_v, rows_v, pos_v, sem):
    wid = lax.axis_index("s") * NC + lax.axis_index("c")

    pltpu.sync_copy(pos_hbm, pos_v)

    def row_body(i, carry):
        gb = wid * ROWS_PER_W + i
        pltpu.sync_copy(inputs_hbm.at[gb], idx_v)
        cp0 = pltpu.async_copy(
            tok_hbm.at[idx_v.at[0]], rows_v.at[pl.ds(0, IDX_CHUNK)], sem)
        cp1 = pltpu.async_copy(
            tok_hbm.at[idx_v.at[1]], rows_v.at[pl.ds(IDX_CHUNK, IDX_CHUNK)], sem)
        cp0.wait()
        cp1.wait()

        def add_body(r, c2):
            for c in range(DIM // LANES):
                sl = pl.ds(c * LANES, LANES)
                plsc.addupdate(rows_v.at[r, sl], pos_v[r, sl])
            return c2

        lax.fori_loop(0, SEQ, add_body, 0)
        pltpu.sync_copy(rows_v, out_hbm.at[gb])
        return carry

    lax.fori_loop(0, ROWS_PER_W, row_body, 0)


@jax.jit
def kernel(inputs, token_table, position_table):
    idx3 = inputs.reshape(BATCH, SEQ // IDX_CHUNK, IDX_CHUNK).astype(jnp.int32)
    mesh = plsc.VectorSubcoreMesh(core_axis_name="c", subcore_axis_name="s")
    run = functools.partial(
        pl.kernel,
        out_type=jax.ShapeDtypeStruct((BATCH, SEQ, DIM), jnp.float32),
        mesh=mesh,
        scratch_types=[
            pltpu.VMEM((SEQ // IDX_CHUNK, IDX_CHUNK), jnp.int32),
            pltpu.VMEM((SEQ, DIM), jnp.float32),
            pltpu.VMEM((SEQ, DIM), jnp.float32),
            pltpu.SemaphoreType.DMA,
        ],
        compiler_params=pltpu.CompilerParams(use_tc_tiling_on_sc=False),
    )(_body)
    return run(idx3, token_table, position_table)


# speedup vs baseline: 1.1172x; 1.1172x over previous
import functools

import jax
import jax.numpy as jnp
from jax import lax
from jax.experimental import pallas as pl
from jax.experimental.pallas import tpu as pltpu
from jax.experimental.pallas import tpu_sc as plsc

NC = 2
NS = 16
LANES = 16
NW = NC * NS

BATCH = 4096
SEQ = 200
DIM = 64
ROWS_PER_W = BATCH // NW
PAIR = 2
IDX_CHUNK = 100
NCHUNK = SEQ // IDX_CHUNK


def _body(inputs_hbm, tok_hbm, pos_hbm, out_hbm, idx_v, rows_v, pos_v,
          sem_g, sem_o):
    wid = lax.axis_index("s") * NC + lax.axis_index("c")
    base = wid * ROWS_PER_W

    pltpu.sync_copy(pos_hbm, pos_v)

    def pair_body(it, carry):
        r0 = base + PAIR * it

        pltpu.sync_copy(inputs_hbm.at[pl.ds(r0, PAIR)], idx_v)

        @pl.when(it > 0)
        def _():
            for b in range(PAIR):
                pltpu.make_async_copy(rows_v.at[b], out_hbm.at[r0], sem_o).wait()

        gathers = []
        for b in range(PAIR):
            for j in range(NCHUNK):
                gathers.append(pltpu.async_copy(
                    tok_hbm.at[idx_v.at[b, j]],
                    rows_v.at[b, pl.ds(j * IDX_CHUNK, IDX_CHUNK)],
                    sem_g))

        for b in range(PAIR):
            for j in range(NCHUNK):
                gathers[NCHUNK * b + j].wait()

            @plsc.parallel_loop(0, SEQ, unroll=4)
            def _(r):
                for c in range(DIM // LANES):
                    sl = pl.ds(c * LANES, LANES)
                    plsc.addupdate(rows_v.at[b, r, sl], pos_v[r, sl])

            pltpu.async_copy(rows_v.at[b], out_hbm.at[r0 + b], sem_o)
        return carry

    lax.fori_loop(0, ROWS_PER_W // PAIR, pair_body, 0)

    for b in range(PAIR):
        pltpu.make_async_copy(rows_v.at[b], out_hbm.at[base], sem_o).wait()


@jax.jit
def kernel(inputs, token_table, position_table):
    idx3 = inputs.reshape(BATCH, NCHUNK, IDX_CHUNK).astype(jnp.int32)
    mesh = plsc.VectorSubcoreMesh(core_axis_name="c", subcore_axis_name="s")
    run = functools.partial(
        pl.kernel,
        out_type=jax.ShapeDtypeStruct((BATCH, SEQ, DIM), jnp.float32),
        mesh=mesh,
        scratch_types=[
            pltpu.VMEM((PAIR, NCHUNK, IDX_CHUNK), jnp.int32),
            pltpu.VMEM((PAIR, SEQ, DIM), jnp.float32),
            pltpu.VMEM((SEQ, DIM), jnp.float32),
            pltpu.SemaphoreType.DMA,
            pltpu.SemaphoreType.DMA,
        ],
        compiler_params=pltpu.CompilerParams(use_tc_tiling_on_sc=False),
    )(_body)
    return run(idx3, token_table, position_table)
